# baseline (speedup 1.0000x reference)
import functools

import jax
import jax.numpy as jnp
from jax import lax
from jax.experimental import pallas as pl
from jax.experimental.pallas import tpu as pltpu
from jax.experimental.pallas import tpu_sc as plsc

_K = 4
_NC = 1 << _K


def _make_sc_gather(v, b, tk):
    info = plsc.get_sparse_core_info()
    ncores, nsub = info.num_cores, info.num_subcores
    nw = ncores * nsub
    rows_per_w = tk // nw
    ch = min(rows_per_w, 16)
    nch = rows_per_w // ch
    mesh = plsc.VectorSubcoreMesh(core_axis_name="c", subcore_axis_name="s")

    @functools.partial(
        pl.kernel,
        out_type=jax.ShapeDtypeStruct((tk, b), jnp.float32),
        mesh=mesh,
        scratch_types=[
            pltpu.VMEM((ch,), jnp.int32),
            pltpu.VMEM((ch, b), jnp.float32),
            pltpu.SemaphoreType.DMA,
        ],
    )
    def gather(xt_hbm, idx_hbm, out_hbm, idx_v, rows_v, sem):
        wid = lax.axis_index("s") * ncores + lax.axis_index("c")
        base = wid * rows_per_w
        for c in range(nch):
            off = base + c * ch
            pltpu.sync_copy(idx_hbm.at[pl.ds(off, ch)], idx_v)
            pltpu.async_copy(xt_hbm.at[idx_v], rows_v, sem).wait()
            pltpu.sync_copy(rows_v, out_hbm.at[pl.ds(off, ch)])

    return gather


def _moebius():
    jj = lax.broadcasted_iota(jnp.int32, (_NC, _NC), 0)
    ss = lax.broadcasted_iota(jnp.int32, (_NC, _NC), 1)
    subset = (jj & ss) == jj
    d = ss ^ jj
    pc = (d & 1) + ((d >> 1) & 1) + ((d >> 2) & 1) + ((d >> 3) & 1)
    sign = (1 - 2 * (pc & 1)).astype(jnp.float32)
    return jnp.where(subset, sign, 0.0)


def _dense_body(g_ref, cpd_ref, out_ref):
    t = g_ref.shape[0] // _K

    g0 = g_ref[0 * t:1 * t, :]
    g1 = g_ref[1 * t:2 * t, :]
    g2 = g_ref[2 * t:3 * t, :]
    g3 = g_ref[3 * t:4 * t, :]

    cp = jnp.dot(cpd_ref[...], _moebius(),
                 preferred_element_type=jnp.float32)

    q3 = g2 * g3
    p3 = g0 * g1

    def r(i):
        return (cp[:, 4 * i + 0:4 * i + 1]
                + cp[:, 4 * i + 1:4 * i + 2] * g3
                + cp[:, 4 * i + 2:4 * i + 3] * g2
                + cp[:, 4 * i + 3:4 * i + 4] * q3)

    inner = r(0) + r(1) * g1 + r(2) * g0 + r(3) * p3
    out_ref[...] = jnp.sum(inner, axis=0, keepdims=True)


def kernel(x, func_vars, cpd):
    b, v = x.shape
    t, k = func_vars.shape
    assert k == _K
    tk = t * k

    xt = x.T
    fv_flat = func_vars.T.reshape(tk).astype(jnp.int32)

    g = _make_sc_gather(v, b, tk)(xt, fv_flat)

    bb = 1024
    out = pl.pallas_call(
        _dense_body,
        grid=(b // bb,),
        in_specs=[
            pl.BlockSpec((tk, bb), lambda i: (0, i)),
            pl.BlockSpec((t, _NC), lambda i: (0, 0)),
        ],
        out_specs=pl.BlockSpec((1, bb), lambda i: (0, i)),
        out_shape=jax.ShapeDtypeStruct((1, b), jnp.float32),
    )(g, cpd)
    return out.reshape(b)
